# baseline (speedup 1.0000x reference)
import functools

import jax
import jax.numpy as jnp
from jax import lax
from jax.experimental import pallas as pl
from jax.experimental.pallas import tpu as pltpu
from jax.experimental.pallas import tpu_sc as plsc

BT = 128


def _gate_body(x_ref, wg_ref, bias_ref, dst_ref, counts_ref):
    logits = lax.dot_general(x_ref[...], wg_ref[...],
                             (((1,), (1,)), ((), ())),
                             preferred_element_type=jnp.float32)
    logits = logits + bias_ref[...]
    probs = jax.nn.softmax(logits, axis=-1)
    t, e = probs.shape
    m = jnp.max(probs, axis=-1, keepdims=True)
    ii = lax.broadcasted_iota(jnp.int32, (t, e), 1)
    te = jnp.min(jnp.where(probs >= m, ii, e), axis=-1)
    oh = (ii == te[:, None]).astype(jnp.float32)
    c = 512
    tri_low = (lax.broadcasted_iota(jnp.int32, (c, c), 1)
               <= lax.broadcasted_iota(jnp.int32, (c, c), 0)).astype(jnp.float32)
    parts = []
    carry = jnp.zeros((1, e), jnp.float32)
    for j in range(t // c):
        blk = oh[j * c:(j + 1) * c]
        pcb = lax.dot_general(tri_low, blk, (((1,), (0,)), ((), ())),
                              preferred_element_type=jnp.float32) + carry
        parts.append(pcb)
        carry = pcb[c - 1:c]
    pc = jnp.concatenate(parts, axis=0)
    counts = pc[t - 1:t]
    cap = jnp.ceil(counts / BT) * BT
    tri = (lax.broadcasted_iota(jnp.int32, (e, e), 0)
           <= lax.broadcasted_iota(jnp.int32, (e, e), 1)).astype(jnp.float32)
    cap_start = lax.dot_general(cap, tri, (((1,), (0,)), ((), ()))) - cap
    dst = jnp.sum((pc - 1.0 + cap_start) * oh, axis=1)
    dst_ref[...] = dst.astype(jnp.int32)
    counts_ref[...] = counts.astype(jnp.int32)


def _gating(flat, wg, bias):
    t, d = flat.shape
    e = wg.shape[0]
    return pl.pallas_call(
        _gate_body,
        out_shape=(jax.ShapeDtypeStruct((t,), jnp.int32),
                   jax.ShapeDtypeStruct((1, e), jnp.int32)),
    )(flat, wg, bias.reshape(1, e))


def _make_row_scatter(n_src, n_out, d):
    nw = 32
    ch = 64
    n_per = n_src // nw
    assert n_src % (nw * ch) == 0
    mesh = plsc.VectorSubcoreMesh(core_axis_name="c", subcore_axis_name="s")

    @functools.partial(
        pl.kernel, mesh=mesh,
        out_type=jax.ShapeDtypeStruct((n_out, d), jnp.float32),
        scratch_types=[
            pltpu.VMEM((ch,), jnp.int32),
            pltpu.VMEM((ch, d), jnp.float32),
            pltpu.SemaphoreType.DMA,
        ],
    )
    def scatter(table_hbm, idx_hbm, out_hbm, idx_v, rows_v, sem):
        wid = lax.axis_index("s") * 2 + lax.axis_index("c")
        base = wid * n_per

        def body(c, carry):
            off = pl.multiple_of(base + c * ch, ch)
            pltpu.sync_copy(idx_hbm.at[pl.ds(off, ch)], idx_v)
            pltpu.sync_copy(table_hbm.at[pl.ds(off, ch)], rows_v)
            pltpu.async_copy(rows_v, out_hbm.at[idx_v], sem).wait()
            return carry

        lax.fori_loop(0, n_per // ch, body, 0)

    return scatter


def _make_row_gather(n_rows, n_idx, d):
    nw = 32
    ch = 64
    n_per = n_idx // nw
    assert n_idx % (nw * ch) == 0
    mesh = plsc.VectorSubcoreMesh(core_axis_name="c", subcore_axis_name="s")

    @functools.partial(
        pl.kernel, mesh=mesh,
        out_type=jax.ShapeDtypeStruct((n_idx, d), jnp.float32),
        scratch_types=[
            pltpu.VMEM((ch,), jnp.int32),
            pltpu.VMEM((ch, d), jnp.float32),
            pltpu.SemaphoreType.DMA,
        ],
    )
    def gather(table_hbm, idx_hbm, out_hbm, idx_v, rows_v, sem):
        wid = lax.axis_index("s") * 2 + lax.axis_index("c")
        base = wid * n_per

        def body(c, carry):
            off = pl.multiple_of(base + c * ch, ch)
            pltpu.sync_copy(idx_hbm.at[pl.ds(off, ch)], idx_v)
            pltpu.async_copy(table_hbm.at[idx_v], rows_v, sem).wait()
            pltpu.sync_copy(rows_v, out_hbm.at[pl.ds(off, ch)])
            return carry

        lax.fori_loop(0, n_per // ch, body, 0)

    return gather


KS = 2


def _ffn_body(be_ref, nu_ref, xs_ref, w1_ref, b1_ref, w2_ref, b2_ref, out_ref):
    k = pl.program_id(0)
    i = pl.program_id(1)

    @pl.when(i < nu_ref[0])
    def _():
        h = lax.dot_general(xs_ref[...], w1_ref[0],
                            (((1,), (1,)), ((), ())),
                            preferred_element_type=jnp.float32)
        h = jnp.maximum(h + b1_ref[0, 0], 0.0)
        part = lax.dot_general(h, w2_ref[0],
                               (((1,), (1,)), ((), ())),
                               preferred_element_type=jnp.float32)
        scale = jnp.where(k == 0, 1.0, 0.0)
        out_ref[0] = part + b2_ref[0] * scale


def _ffn(xs, w1, b1, w2, b2, block_expert, n_used):
    p, d = xs.shape
    e, d_ff, _ = w1.shape
    nblk = p // BT
    dff2 = d_ff // KS
    b1r = b1.reshape(e, KS, 1, dff2)
    b2r = b2.reshape(e, 1, d)
    grid_spec = pltpu.PrefetchScalarGridSpec(
        num_scalar_prefetch=2,
        grid=(KS, nblk),
        in_specs=[
            pl.BlockSpec((BT, d), lambda k, i, be, nu: (i, 0)),
            pl.BlockSpec((1, dff2, d), lambda k, i, be, nu: (be[i], k, 0)),
            pl.BlockSpec((1, 1, 1, dff2), lambda k, i, be, nu: (be[i], k, 0, 0)),
            pl.BlockSpec((1, d, dff2), lambda k, i, be, nu: (be[i], 0, k)),
            pl.BlockSpec((1, 1, d), lambda k, i, be, nu: (be[i], 0, 0)),
        ],
        out_specs=pl.BlockSpec((1, BT, d), lambda k, i, be, nu: (k, i, 0)),
    )
    return pl.pallas_call(
        _ffn_body,
        grid_spec=grid_spec,
        out_shape=jax.ShapeDtypeStruct((KS, p, d), jnp.float32),
        compiler_params=pltpu.CompilerParams(
            dimension_semantics=("arbitrary", "arbitrary")),
    )(block_expert, n_used, xs, w1, b1r, w2, b2r)


def kernel(x, Wg, bg, W1, b1, W2, b2, expert_bias):
    b, s, d = x.shape
    e, d_ff, _ = W1.shape
    t = b * s
    p = t + e * BT
    flat = x.reshape(t, d)

    dst, counts = _gating(flat, Wg, bg + expert_bias)

    cap = ((counts[0] + BT - 1) // BT) * BT
    cap_cum = jnp.cumsum(cap)
    nblk = p // BT
    blk_off = jnp.arange(nblk, dtype=jnp.int32) * BT
    total_cap = cap_cum[-1]
    block_expert = jnp.searchsorted(
        cap_cum, jnp.minimum(blk_off, total_cap - 1), side="right"
    ).astype(jnp.int32)
    n_used = (total_cap // BT).astype(jnp.int32).reshape(1)

    xs = _make_row_scatter(t, p, d)(flat, dst)
    ys = _ffn(xs, W1, b1, W2, b2, block_expert, n_used)
    out = _make_row_gather(p, t, d)(ys[0] + ys[1], dst)
    return out.reshape(b, s, d)


# speedup vs baseline: 1.4406x; 1.4406x over previous
import functools

import jax
import jax.numpy as jnp
from jax import lax
from jax.experimental import pallas as pl
from jax.experimental.pallas import tpu as pltpu
from jax.experimental.pallas import tpu_sc as plsc

BT = 256


def _gate_body(x_ref, wg_ref, bias_ref, dst_ref, counts_ref):
    logits = lax.dot_general(x_ref[...], wg_ref[...],
                             (((1,), (1,)), ((), ())),
                             preferred_element_type=jnp.float32)
    logits = logits + bias_ref[...]
    probs = jax.nn.softmax(logits, axis=-1)
    t, e = probs.shape
    m = jnp.max(probs, axis=-1, keepdims=True)
    ii = lax.broadcasted_iota(jnp.int32, (t, e), 1)
    te = jnp.min(jnp.where(probs >= m, ii, e), axis=-1)
    oh = (ii == te[:, None]).astype(jnp.float32)
    c = 512
    tri_low = (lax.broadcasted_iota(jnp.int32, (c, c), 1)
               <= lax.broadcasted_iota(jnp.int32, (c, c), 0)).astype(jnp.float32)
    parts = []
    carry = jnp.zeros((1, e), jnp.float32)
    for j in range(t // c):
        blk = oh[j * c:(j + 1) * c]
        pcb = lax.dot_general(tri_low, blk, (((1,), (0,)), ((), ())),
                              preferred_element_type=jnp.float32) + carry
        parts.append(pcb)
        carry = pcb[c - 1:c]
    pc = jnp.concatenate(parts, axis=0)
    counts = pc[t - 1:t]
    cap = jnp.ceil(counts / BT) * BT
    tri = (lax.broadcasted_iota(jnp.int32, (e, e), 0)
           <= lax.broadcasted_iota(jnp.int32, (e, e), 1)).astype(jnp.float32)
    cap_start = lax.dot_general(cap, tri, (((1,), (0,)), ((), ()))) - cap
    dst = jnp.sum((pc - 1.0 + cap_start) * oh, axis=1)
    dst_ref[...] = dst.astype(jnp.int32)
    counts_ref[...] = counts.astype(jnp.int32)


def _gating(flat, wg, bias):
    t, d = flat.shape
    e = wg.shape[0]
    return pl.pallas_call(
        _gate_body,
        out_shape=(jax.ShapeDtypeStruct((t,), jnp.int32),
                   jax.ShapeDtypeStruct((1, e), jnp.int32)),
    )(flat, wg, bias.reshape(1, e))


def _make_row_scatter(n_src, n_out, d):
    nw = 32
    ch = 64
    n_per = n_src // nw
    assert n_src % (nw * ch) == 0
    mesh = plsc.VectorSubcoreMesh(core_axis_name="c", subcore_axis_name="s")

    @functools.partial(
        pl.kernel, mesh=mesh,
        out_type=jax.ShapeDtypeStruct((n_out, d), jnp.float32),
        scratch_types=[
            pltpu.VMEM((ch,), jnp.int32),
            pltpu.VMEM((ch, d), jnp.float32),
            pltpu.SemaphoreType.DMA,
        ],
    )
    def scatter(table_hbm, idx_hbm, out_hbm, idx_v, rows_v, sem):
        wid = lax.axis_index("s") * 2 + lax.axis_index("c")
        base = wid * n_per

        def body(c, carry):
            off = pl.multiple_of(base + c * ch, ch)
            pltpu.sync_copy(idx_hbm.at[pl.ds(off, ch)], idx_v)
            pltpu.sync_copy(table_hbm.at[pl.ds(off, ch)], rows_v)
            pltpu.async_copy(rows_v, out_hbm.at[idx_v], sem).wait()
            return carry

        lax.fori_loop(0, n_per // ch, body, 0)

    return scatter


def _make_row_gather(n_rows, n_idx, d):
    nw = 32
    ch = 64
    n_per = n_idx // nw
    assert n_idx % (nw * ch) == 0
    mesh = plsc.VectorSubcoreMesh(core_axis_name="c", subcore_axis_name="s")

    @functools.partial(
        pl.kernel, mesh=mesh,
        out_type=jax.ShapeDtypeStruct((n_idx, d), jnp.float32),
        scratch_types=[
            pltpu.VMEM((ch,), jnp.int32),
            pltpu.VMEM((ch, d), jnp.float32),
            pltpu.SemaphoreType.DMA,
        ],
    )
    def gather(table_hbm, idx_hbm, out_hbm, idx_v, rows_v, sem):
        wid = lax.axis_index("s") * 2 + lax.axis_index("c")
        base = wid * n_per

        def body(c, carry):
            off = pl.multiple_of(base + c * ch, ch)
            pltpu.sync_copy(idx_hbm.at[pl.ds(off, ch)], idx_v)
            pltpu.async_copy(table_hbm.at[idx_v], rows_v, sem).wait()
            pltpu.sync_copy(rows_v, out_hbm.at[pl.ds(off, ch)])
            return carry

        lax.fori_loop(0, n_per // ch, body, 0)

    return gather


KS = 2


def _ffn_body(be_ref, nu_ref, xs_ref, w1_ref, b1_ref, w2_ref, b2_ref, out_ref):
    k = pl.program_id(0)
    i = pl.program_id(1)

    @pl.when(i < nu_ref[0])
    def _():
        h = lax.dot_general(xs_ref[...], w1_ref[0],
                            (((1,), (1,)), ((), ())),
                            preferred_element_type=jnp.float32)
        h = jnp.maximum(h + b1_ref[0, 0], 0.0)
        part = lax.dot_general(h, w2_ref[0],
                               (((1,), (1,)), ((), ())),
                               preferred_element_type=jnp.float32)
        scale = jnp.where(k == 0, 1.0, 0.0)
        out_ref[0] = part + b2_ref[0] * scale


def _ffn(xs, w1, b1, w2, b2, block_expert, n_used):
    p, d = xs.shape
    e, d_ff, _ = w1.shape
    nblk = p // BT
    dff2 = d_ff // KS
    b1r = b1.reshape(e, KS, 1, dff2)
    b2r = b2.reshape(e, 1, d)
    grid_spec = pltpu.PrefetchScalarGridSpec(
        num_scalar_prefetch=2,
        grid=(KS, nblk),
        in_specs=[
            pl.BlockSpec((BT, d), lambda k, i, be, nu: (i, 0)),
            pl.BlockSpec((1, dff2, d), lambda k, i, be, nu: (be[i], k, 0)),
            pl.BlockSpec((1, 1, 1, dff2), lambda k, i, be, nu: (be[i], k, 0, 0)),
            pl.BlockSpec((1, d, dff2), lambda k, i, be, nu: (be[i], 0, k)),
            pl.BlockSpec((1, 1, d), lambda k, i, be, nu: (be[i], 0, 0)),
        ],
        out_specs=pl.BlockSpec((1, BT, d), lambda k, i, be, nu: (k, i, 0)),
    )
    return pl.pallas_call(
        _ffn_body,
        grid_spec=grid_spec,
        out_shape=jax.ShapeDtypeStruct((KS, p, d), jnp.float32),
        compiler_params=pltpu.CompilerParams(
            dimension_semantics=("arbitrary", "arbitrary")),
    )(block_expert, n_used, xs, w1, b1r, w2, b2r)


def kernel(x, Wg, bg, W1, b1, W2, b2, expert_bias):
    b, s, d = x.shape
    e, d_ff, _ = W1.shape
    t = b * s
    p = t + e * BT
    flat = x.reshape(t, d)

    dst, counts = _gating(flat, Wg, bg + expert_bias)

    cap = ((counts[0] + BT - 1) // BT) * BT
    cap_cum = jnp.cumsum(cap)
    nblk = p // BT
    blk_off = jnp.arange(nblk, dtype=jnp.int32) * BT
    total_cap = cap_cum[-1]
    block_expert = jnp.searchsorted(
        cap_cum, jnp.minimum(blk_off, total_cap - 1), side="right"
    ).astype(jnp.int32)
    n_used = (total_cap // BT).astype(jnp.int32).reshape(1)

    xs = _make_row_scatter(t, p, d)(flat, dst)
    ys = _ffn(xs, W1, b1, W2, b2, block_expert, n_used)
    out = _make_row_gather(p, t, d)(ys[0] + ys[1], dst)
    return out.reshape(b, s, d)
